# baseline (speedup 1.0000x reference)
import functools

import jax
import jax.numpy as jnp
from jax import lax
from jax.experimental import pallas as pl
from jax.experimental.pallas import tpu as pltpu
from jax.experimental.pallas import tpu_sc as plsc

N_BINS = 15
N_ROWS = 1_000_000
N_CLS = 64
NW = 32
CHUNK = 320
NCHUNK = 70
ROWS_W = CHUNK * NCHUNK
SC_ROWS = ROWS_W * NW
TC_ROWS = N_ROWS - SC_ROWS
GROUPS = CHUNK // 16
TC_BLOCK = 7080
TC_GRID = TC_ROWS // TC_BLOCK


def _iota16():
    return lax.iota(jnp.int32, 16)


def _process_group(lbuf, labbuf, hists, g):
    cnt_h, acc_h, conf_h = hists
    lanes = _iota16()
    rows = g * 16 + lanes
    ms = [jnp.full((16,), -jnp.inf, dtype=jnp.float32) for _ in range(4)]
    ss = [jnp.zeros((16,), dtype=jnp.float32) for _ in range(4)]
    for j in range(N_CLS):
        v = plsc.load_gather(lbuf, [rows, (lanes + j) & (N_CLS - 1)])
        ss[j % 4] = ss[j % 4] + jnp.exp(v)
        ms[j % 4] = jnp.maximum(ms[j % 4], v)
    s = (ss[0] + ss[1]) + (ss[2] + ss[3])
    m = jnp.maximum(jnp.maximum(ms[0], ms[1]), jnp.maximum(ms[2], ms[3]))
    conf = jnp.exp(m) / s

    lab = plsc.load_gather(labbuf, [rows])
    v_lab = plsc.load_gather(lbuf, [rows, lab])
    acc = jnp.where(v_lab == m, 1.0, 0.0).astype(jnp.float32)

    binf = conf * jnp.float32(N_BINS)
    bi = binf.astype(jnp.int32)
    exact = (bi.astype(jnp.float32) == binf).astype(jnp.int32)
    bi = jnp.minimum(bi - exact, N_BINS - 1)

    slots = lanes * 16 + bi
    ones = jnp.ones((16,), dtype=jnp.float32)
    plsc.addupdate_scatter(cnt_h, [slots], ones)
    plsc.addupdate_scatter(acc_h, [slots], acc)
    plsc.addupdate_scatter(conf_h, [slots], conf)


def _sc_partials(logits, labels):
    mesh = plsc.VectorSubcoreMesh(core_axis_name="c", subcore_axis_name="s")

    @functools.partial(
        pl.kernel,
        mesh=mesh,
        out_type=jax.ShapeDtypeStruct((NW, 48), jnp.float32),
        compiler_params=pltpu.CompilerParams(needs_layout_passes=False),
        scratch_types=[
            pltpu.VMEM((CHUNK, N_CLS), jnp.float32),
            pltpu.VMEM((CHUNK, N_CLS), jnp.float32),
            pltpu.VMEM((CHUNK,), jnp.int32),
            pltpu.VMEM((CHUNK,), jnp.int32),
            pltpu.VMEM((256,), jnp.float32),
            pltpu.VMEM((256,), jnp.float32),
            pltpu.VMEM((256,), jnp.float32),
            pltpu.VMEM((48,), jnp.float32),
            pltpu.SemaphoreType.DMA,
            pltpu.SemaphoreType.DMA,
            pltpu.SemaphoreType.DMA,
            pltpu.SemaphoreType.DMA,
        ],
    )
    def body(logits_hbm, labels_hbm, out_hbm, lbuf0, lbuf1, labb0, labb1,
             cnt_h, acc_h, conf_h, stage, sg0, sg1, sb0, sb1):
        wid = lax.axis_index("s") * 2 + lax.axis_index("c")
        row0 = wid * ROWS_W
        lanes = _iota16()
        z16 = jnp.zeros((16,), dtype=jnp.float32)
        for k in range(16):
            cnt_h[pl.ds(k * 16, 16)] = z16
            acc_h[pl.ds(k * 16, 16)] = z16
            conf_h[pl.ds(k * 16, 16)] = z16
        hists = (cnt_h, acc_h, conf_h)

        def dma_logits(c, buf, sem):
            return pltpu.make_async_copy(
                logits_hbm.at[pl.ds(row0 + c * CHUNK, CHUNK)], buf, sem)

        def dma_labels(c, buf, sem):
            return pltpu.make_async_copy(
                labels_hbm.at[pl.ds(row0 + c * CHUNK, CHUNK)], buf, sem)

        def process(lbuf, labbuf):
            def g_body(g, carry):
                _process_group(lbuf, labbuf, hists, g)
                return carry
            lax.fori_loop(0, GROUPS, g_body, 0)

        dma_logits(0, lbuf0, sg0).start()
        dma_labels(0, labb0, sb0).start()

        def pair_body(c2, carry):
            c = c2 * 2
            dma_logits(c + 1, lbuf1, sg1).start()
            dma_labels(c + 1, labb1, sb1).start()
            dma_logits(c, lbuf0, sg0).wait()
            dma_labels(c, labb0, sb0).wait()
            process(lbuf0, labb0)

            @pl.when(c2 < (NCHUNK // 2) - 1)
            def _next():
                dma_logits(c + 2, lbuf0, sg0).start()
                dma_labels(c + 2, labb0, sb0).start()
            dma_logits(c + 1, lbuf1, sg1).wait()
            dma_labels(c + 1, labb1, sb1).wait()
            process(lbuf1, labb1)
            return carry
        lax.fori_loop(0, NCHUNK // 2, pair_body, 0)

        for h_idx, h in enumerate(hists):
            tot = z16
            for lane in range(16):
                tot = tot + plsc.load_gather(h, [lane * 16 + lanes])
            stage[pl.ds(h_idx * 16, 16)] = tot
        pltpu.sync_copy(stage, out_hbm.at[wid])

    return body(logits, labels)


def _tc_partials_body(logits_ref, labels_ref, out_ref, acc_ref):
    i = pl.program_id(0)

    @pl.when(i == 0)
    def _init():
        acc_ref[...] = jnp.zeros_like(acc_ref)

    l = logits_ref[...]
    lab = labels_ref[0]
    b, c = l.shape
    m = jnp.max(l, axis=1, keepdims=True)
    z = jnp.sum(jnp.exp(l - m), axis=1, keepdims=True)
    conf = 1.0 / z
    iota = lax.broadcasted_iota(jnp.int32, (b, c), 1)
    pred = jnp.min(jnp.where(l == m, iota, c), axis=1, keepdims=True)
    acc = (pred == lab).astype(jnp.float32)

    bidx = lax.broadcasted_iota(jnp.int32, (1, N_BINS), 1).astype(jnp.float32)
    lows = bidx / N_BINS
    highs = (bidx + 1.0) / N_BINS
    in_bin = ((conf > lows) & (conf <= highs)).astype(jnp.float32)

    acc_ref[0:1, 0:N_BINS] += jnp.sum(in_bin, axis=0, keepdims=True)
    acc_ref[1:2, 0:N_BINS] += jnp.sum(in_bin * acc, axis=0, keepdims=True)
    acc_ref[2:3, 0:N_BINS] += jnp.sum(in_bin * conf, axis=0, keepdims=True)

    @pl.when(i == TC_GRID - 1)
    def _emit():
        out_ref[0:1, 0:16] = acc_ref[0:1, 0:16]
        out_ref[0:1, 16:32] = acc_ref[1:2, 0:16]
        out_ref[0:1, 32:48] = acc_ref[2:3, 0:16]


def _combine_body(p_ref, q_ref, out_ref):
    p = p_ref[...]
    q = q_ref[...]
    cnt = (jnp.sum(p[:, 0:N_BINS], axis=0, keepdims=True)
           + q[0:1, 0:N_BINS])
    asum = (jnp.sum(p[:, 16:16 + N_BINS], axis=0, keepdims=True)
            + q[0:1, 16:16 + N_BINS])
    csum = (jnp.sum(p[:, 32:32 + N_BINS], axis=0, keepdims=True)
            + q[0:1, 32:32 + N_BINS])
    prop = cnt / N_ROWS
    safe = jnp.maximum(cnt, 1.0)
    nonempty = (cnt > 0).astype(jnp.float32)
    per_bin = jnp.abs(csum / safe - asum / safe) * prop * nonempty
    out_ref[...] = jnp.sum(per_bin, axis=1, keepdims=True)


def kernel(logits, labels):
    labels = labels.astype(jnp.int32)
    partials = _sc_partials(logits, labels)
    tail_logits = logits[SC_ROWS:]
    tail_labels = labels[SC_ROWS:].reshape(TC_GRID, TC_BLOCK, 1)
    tc_part = pl.pallas_call(
        _tc_partials_body,
        grid=(TC_GRID,),
        in_specs=[
            pl.BlockSpec((TC_BLOCK, N_CLS), lambda i: (i, 0)),
            pl.BlockSpec((1, TC_BLOCK, 1), lambda i: (i, 0, 0)),
        ],
        out_specs=pl.BlockSpec((1, 48), lambda i: (0, 0)),
        out_shape=jax.ShapeDtypeStruct((1, 48), jnp.float32),
        scratch_shapes=[pltpu.VMEM((8, 128), jnp.float32)],
    )(tail_logits, tail_labels)
    out = pl.pallas_call(
        _combine_body,
        out_shape=jax.ShapeDtypeStruct((1, 1), jnp.float32),
    )(partials, tc_part)
    return out.reshape(1)


# speedup vs baseline: 1.4091x; 1.4091x over previous
import functools

import jax
import jax.numpy as jnp
from jax import lax
from jax.experimental import pallas as pl
from jax.experimental.pallas import tpu as pltpu
from jax.experimental.pallas import tpu_sc as plsc

N_BINS = 15
N_ROWS = 1_000_000
N_CLS = 64
NW = 32
CHUNK = 480
NCHUNK = 64
ROWS_W = CHUNK * NCHUNK
SC_ROWS = ROWS_W * NW
TC_ROWS = N_ROWS - SC_ROWS
GROUPS = CHUNK // 16
TC_BLOCK = TC_ROWS
TC_GRID = 1


def _iota16():
    return lax.iota(jnp.int32, 16)


def _process_group(lbuf, labbuf, hists, g):
    cnt_h, acc_h, conf_h = hists
    lanes = _iota16()
    rows = g * 16 + lanes
    ms = [jnp.full((16,), -jnp.inf, dtype=jnp.float32) for _ in range(4)]
    ss = [jnp.zeros((16,), dtype=jnp.float32) for _ in range(4)]
    for j in range(N_CLS):
        v = plsc.load_gather(lbuf, [rows, (lanes + j) & (N_CLS - 1)])
        ss[j % 4] = ss[j % 4] + jnp.exp(v)
        ms[j % 4] = jnp.maximum(ms[j % 4], v)
    s = (ss[0] + ss[1]) + (ss[2] + ss[3])
    m = jnp.maximum(jnp.maximum(ms[0], ms[1]), jnp.maximum(ms[2], ms[3]))
    conf = jnp.exp(m) / s

    lab = plsc.load_gather(labbuf, [rows])
    v_lab = plsc.load_gather(lbuf, [rows, lab])
    acc = jnp.where(v_lab == m, 1.0, 0.0).astype(jnp.float32)

    binf = conf * jnp.float32(N_BINS)
    bi = binf.astype(jnp.int32)
    exact = (bi.astype(jnp.float32) == binf).astype(jnp.int32)
    bi = jnp.minimum(bi - exact, N_BINS - 1)

    slots = lanes * 16 + bi
    ones = jnp.ones((16,), dtype=jnp.float32)
    plsc.addupdate_scatter(cnt_h, [slots], ones)
    plsc.addupdate_scatter(acc_h, [slots], acc)
    plsc.addupdate_scatter(conf_h, [slots], conf)


def _sc_partials(logits, labels):
    mesh = plsc.VectorSubcoreMesh(core_axis_name="c", subcore_axis_name="s")

    @functools.partial(
        pl.kernel,
        mesh=mesh,
        out_type=jax.ShapeDtypeStruct((NW, 48), jnp.float32),
        compiler_params=pltpu.CompilerParams(needs_layout_passes=False),
        scratch_types=[
            pltpu.VMEM((CHUNK, N_CLS), jnp.float32),
            pltpu.VMEM((CHUNK, N_CLS), jnp.float32),
            pltpu.VMEM((CHUNK,), jnp.int32),
            pltpu.VMEM((CHUNK,), jnp.int32),
            pltpu.VMEM((256,), jnp.float32),
            pltpu.VMEM((256,), jnp.float32),
            pltpu.VMEM((256,), jnp.float32),
            pltpu.VMEM((48,), jnp.float32),
            pltpu.SemaphoreType.DMA,
            pltpu.SemaphoreType.DMA,
            pltpu.SemaphoreType.DMA,
            pltpu.SemaphoreType.DMA,
        ],
    )
    def body(logits_hbm, labels_hbm, out_hbm, lbuf0, lbuf1, labb0, labb1,
             cnt_h, acc_h, conf_h, stage, sg0, sg1, sb0, sb1):
        wid = lax.axis_index("s") * 2 + lax.axis_index("c")
        row0 = wid * ROWS_W
        lanes = _iota16()
        z16 = jnp.zeros((16,), dtype=jnp.float32)
        for k in range(16):
            cnt_h[pl.ds(k * 16, 16)] = z16
            acc_h[pl.ds(k * 16, 16)] = z16
            conf_h[pl.ds(k * 16, 16)] = z16
        hists = (cnt_h, acc_h, conf_h)

        def dma_logits(c, buf, sem):
            return pltpu.make_async_copy(
                logits_hbm.at[pl.ds(row0 + c * CHUNK, CHUNK)], buf, sem)

        def dma_labels(c, buf, sem):
            return pltpu.make_async_copy(
                labels_hbm.at[pl.ds(row0 + c * CHUNK, CHUNK)], buf, sem)

        def process(lbuf, labbuf):
            def g_body(g, carry):
                _process_group(lbuf, labbuf, hists, g)
                return carry
            lax.fori_loop(0, GROUPS, g_body, 0)

        dma_logits(0, lbuf0, sg0).start()
        dma_labels(0, labb0, sb0).start()

        def pair_body(c2, carry):
            c = c2 * 2
            dma_logits(c + 1, lbuf1, sg1).start()
            dma_labels(c + 1, labb1, sb1).start()
            dma_logits(c, lbuf0, sg0).wait()
            dma_labels(c, labb0, sb0).wait()
            process(lbuf0, labb0)

            @pl.when(c2 < (NCHUNK // 2) - 1)
            def _next():
                dma_logits(c + 2, lbuf0, sg0).start()
                dma_labels(c + 2, labb0, sb0).start()
            dma_logits(c + 1, lbuf1, sg1).wait()
            dma_labels(c + 1, labb1, sb1).wait()
            process(lbuf1, labb1)
            return carry
        lax.fori_loop(0, NCHUNK // 2, pair_body, 0)

        for h_idx, h in enumerate(hists):
            tot = z16
            for lane in range(16):
                tot = tot + plsc.load_gather(h, [lane * 16 + lanes])
            stage[pl.ds(h_idx * 16, 16)] = tot
        pltpu.sync_copy(stage, out_hbm.at[wid])

    return body(logits, labels)


def _tc_partials_body(logits_ref, labels_ref, out_ref, acc_ref):
    i = pl.program_id(0)

    @pl.when(i == 0)
    def _init():
        acc_ref[...] = jnp.zeros_like(acc_ref)

    l = logits_ref[...]
    lab = labels_ref[0]
    b, c = l.shape
    m = jnp.max(l, axis=1, keepdims=True)
    z = jnp.sum(jnp.exp(l - m), axis=1, keepdims=True)
    conf = 1.0 / z
    iota = lax.broadcasted_iota(jnp.int32, (b, c), 1)
    pred = jnp.min(jnp.where(l == m, iota, c), axis=1, keepdims=True)
    acc = (pred == lab).astype(jnp.float32)

    bidx = lax.broadcasted_iota(jnp.int32, (1, N_BINS), 1).astype(jnp.float32)
    lows = bidx / N_BINS
    highs = (bidx + 1.0) / N_BINS
    in_bin = ((conf > lows) & (conf <= highs)).astype(jnp.float32)

    acc_ref[0:1, 0:N_BINS] += jnp.sum(in_bin, axis=0, keepdims=True)
    acc_ref[1:2, 0:N_BINS] += jnp.sum(in_bin * acc, axis=0, keepdims=True)
    acc_ref[2:3, 0:N_BINS] += jnp.sum(in_bin * conf, axis=0, keepdims=True)

    @pl.when(i == TC_GRID - 1)
    def _emit():
        out_ref[0:1, 0:16] = acc_ref[0:1, 0:16]
        out_ref[0:1, 16:32] = acc_ref[1:2, 0:16]
        out_ref[0:1, 32:48] = acc_ref[2:3, 0:16]


def _combine_body(p_ref, q_ref, out_ref):
    p = p_ref[...]
    q = q_ref[...]
    cnt = (jnp.sum(p[:, 0:N_BINS], axis=0, keepdims=True)
           + q[0:1, 0:N_BINS])
    asum = (jnp.sum(p[:, 16:16 + N_BINS], axis=0, keepdims=True)
            + q[0:1, 16:16 + N_BINS])
    csum = (jnp.sum(p[:, 32:32 + N_BINS], axis=0, keepdims=True)
            + q[0:1, 32:32 + N_BINS])
    prop = cnt / N_ROWS
    safe = jnp.maximum(cnt, 1.0)
    nonempty = (cnt > 0).astype(jnp.float32)
    per_bin = jnp.abs(csum / safe - asum / safe) * prop * nonempty
    out_ref[...] = jnp.sum(per_bin, axis=1, keepdims=True)


def kernel(logits, labels):
    labels = labels.astype(jnp.int32)
    partials = _sc_partials(logits, labels)
    tail_logits = logits[SC_ROWS:]
    tail_labels = labels[SC_ROWS:].reshape(TC_GRID, TC_BLOCK, 1)
    tc_part = pl.pallas_call(
        _tc_partials_body,
        grid=(TC_GRID,),
        in_specs=[
            pl.BlockSpec((TC_BLOCK, N_CLS), lambda i: (i, 0)),
            pl.BlockSpec((1, TC_BLOCK, 1), lambda i: (i, 0, 0)),
        ],
        out_specs=pl.BlockSpec((1, 48), lambda i: (0, 0)),
        out_shape=jax.ShapeDtypeStruct((1, 48), jnp.float32),
        scratch_shapes=[pltpu.VMEM((8, 128), jnp.float32)],
    )(tail_logits, tail_labels)
    out = pl.pallas_call(
        _combine_body,
        out_shape=jax.ShapeDtypeStruct((1, 1), jnp.float32),
    )(partials, tc_part)
    return out.reshape(1)
